# Initial kernel scaffold; baseline (speedup 1.0000x reference)
import functools

import jax
import jax.numpy as jnp
from jax import lax
from jax.experimental import pallas as pl
from jax.experimental.pallas import tpu as pltpu
from jax.experimental.pallas import tpu_sc as plsc

B = 16384
L = 200
E = 18
NC = 2
NS = 16
NW = NC * NS
BPW = B // NW
POS = BPW * L
CH = 2048
CHR = CH // 128
NCH = POS // CH

_mesh = plsc.VectorSubcoreMesh(
    core_axis_name="c", subcore_axis_name="s", num_cores=NC, num_subcores=NS)

_f32 = jnp.float32
_i32 = jnp.int32


@functools.partial(
    pl.kernel,
    out_type=[jax.ShapeDtypeStruct((B, E), _f32)] * 5,
    mesh=_mesh,
    scratch_types=[
        pltpu.VMEM((CHR, 128), _i32),
        pltpu.VMEM((CHR, 128), _i32),
        pltpu.VMEM((CH, E), _f32),
        pltpu.VMEM((CH, E), _f32),
        pltpu.VMEM((CHR, 128), _i32),
        pltpu.VMEM((4, 128), _i32),
        pltpu.VMEM((BPW, E), _f32),
        pltpu.VMEM((BPW, E), _f32),
        pltpu.VMEM_SHARED((NS * BPW, E), _f32),
        pltpu.VMEM_SHARED((NS * BPW, E), _f32),
    ],
)
def _sc_embed(uid_i, mid_i, cat_i, mh_i, ch_i, uid_t, mid_t, cat_t,
              uid_o, mid_o, cat_o, mids_o, cats_o,
              idx_m, idx_c, row_m, row_c, b_idx, sidx, srow, zbuf,
              acc_m, acc_c):
  c = lax.axis_index("c")
  s = lax.axis_index("s")
  wid = s * NC + c
  base = wid * BPW
  accbase = s * BPW
  iota16 = lax.iota(_i32, 16)
  z16 = jnp.zeros((16,), _f32)

  def _zero(i, carry):
    zbuf[i, pl.ds(0, 16)] = z16
    zbuf[i, pl.ds(2, 16)] = z16
    return carry
  lax.fori_loop(0, BPW, _zero, 0)
  pltpu.sync_copy(zbuf, acc_m.at[pl.ds(accbase, BPW)])
  pltpu.sync_copy(zbuf, acc_c.at[pl.ds(accbase, BPW)])

  for idx2d, table, out in ((uid_i, uid_t, uid_o),
                            (mid_i, mid_t, mid_o),
                            (cat_i, cat_t, cat_o)):
    pltpu.sync_copy(idx2d.at[pl.ds(wid * (BPW // 128), BPW // 128)], sidx)
    pltpu.sync_copy(table.at[sidx], srow)
    pltpu.sync_copy(srow, out.at[pl.ds(base, BPW)])

  rowbase = wid * (POS // 128)

  def _chunk(ci, carry):
    pltpu.sync_copy(mh_i.at[pl.ds(rowbase + ci * CHR, CHR)], idx_m)
    pltpu.sync_copy(ch_i.at[pl.ds(rowbase + ci * CHR, CHR)], idx_c)
    pltpu.sync_copy(mid_t.at[idx_m], row_m)
    pltpu.sync_copy(cat_t.at[idx_c], row_c)
    pbase = ci * CH
    for j in range(CHR):
      for k in range(8):
        p = pbase + j * 128 + k * 16
        b_idx[j, pl.ds(k * 16, 16)] = (p + iota16) // L + accbase
    pltpu.sync_copy(row_m, acc_m.at[b_idx], add=True)
    pltpu.sync_copy(row_c, acc_c.at[b_idx], add=True)
    return carry
  lax.fori_loop(0, NCH, _chunk, 0)

  pltpu.sync_copy(acc_m.at[pl.ds(accbase, BPW)], mids_o.at[pl.ds(base, BPW)])
  pltpu.sync_copy(acc_c.at[pl.ds(accbase, BPW)], cats_o.at[pl.ds(base, BPW)])


MB = 512


def _mlp_body(u, m, c, ms, cs, w1, b1, w2, b2, w3, b3, o):
  inp = jnp.concatenate([u[...], m[...], c[...], ms[...], cs[...]], axis=1)
  h = jnp.dot(inp, w1[...], preferred_element_type=_f32) + b1[...]
  h = jnp.maximum(h, 0.0)
  h = jnp.dot(h, w2[...], preferred_element_type=_f32) + b2[...]
  h = jnp.maximum(h, 0.0)
  o[...] = jnp.dot(h, w3[...], preferred_element_type=_f32) + b3[...]


def _mlp(u, m, c, ms, cs, w1, b1, w2, b2, w3, b3):
  piece = pl.BlockSpec((MB, E), lambda i: (i, 0))
  full = lambda a: pl.BlockSpec(a.shape, lambda i: (0,) * a.ndim)
  return pl.pallas_call(
      _mlp_body,
      grid=(B // MB,),
      in_specs=[piece] * 5 + [full(w1), full(b1), full(w2), full(b2),
                              full(w3), full(b3)],
      out_specs=pl.BlockSpec((MB, 1), lambda i: (i, 0)),
      out_shape=jax.ShapeDtypeStruct((B, 1), _f32),
  )(u, m, c, ms, cs, w1, b1, w2, b2, w3, b3)


def kernel(uid_batch_ph, mid_batch_ph, mid_his_batch_ph, cat_batch_ph,
           cat_his_batch_ph, mask, seq_len_ph, target_ph, lr,
           uid_table, mid_table, cat_table, W1, b1, W2, b2, W3, b3):
  uid_i = uid_batch_ph.astype(_i32).reshape(B // 128, 128)
  mid_i = mid_batch_ph.astype(_i32).reshape(B // 128, 128)
  cat_i = cat_batch_ph.astype(_i32).reshape(B // 128, 128)
  mh_i = mid_his_batch_ph.astype(_i32).reshape(B * L // 128, 128)
  ch_i = cat_his_batch_ph.astype(_i32).reshape(B * L // 128, 128)

  uid_e, mid_e, cat_e, mids, cats = _sc_embed(
      uid_i, mid_i, cat_i, mh_i, ch_i,
      uid_table.astype(_f32), mid_table.astype(_f32), cat_table.astype(_f32))

  return _mlp(uid_e, mid_e, cat_e, mids, cats,
              W1, b1.reshape(1, 200), W2, b2.reshape(1, 80),
              W3, b3.reshape(1, 1))


# speedup vs baseline: 7.2650x; 7.2650x over previous
import functools

import jax
import jax.numpy as jnp
from jax import lax
from jax.experimental import pallas as pl
from jax.experimental.pallas import tpu as pltpu
from jax.experimental.pallas import tpu_sc as plsc

B = 16384
L = 200
E = 18
EP = 32
NC = 2
NS = 16
NW = NC * NS
BPW = B // NW
POS = BPW * L
NP = 2
NPP = BPW // NP
POSH = POS // NP
CH = 1024
NCH = POSH // CH

_mesh = plsc.VectorSubcoreMesh(
    core_axis_name="c", subcore_axis_name="s", num_cores=NC, num_subcores=NS)

_f32 = jnp.float32
_i32 = jnp.int32


@functools.partial(
    pl.kernel,
    out_type=[pltpu.HBM((B, EP), _f32)] * 5,
    mesh=_mesh,
    scratch_types=[
        pltpu.VMEM((CH,), _i32),
        pltpu.VMEM((CH,), _i32),
        pltpu.VMEM((CH, EP), _f32),
        pltpu.VMEM((CH, EP), _f32),
        pltpu.VMEM((CH,), _i32),
        pltpu.VMEM((BPW,), _i32),
        pltpu.VMEM((BPW, EP), _f32),
        pltpu.VMEM_SHARED((NS * NPP, EP), _f32),
        pltpu.VMEM_SHARED((NS * NPP, EP), _f32),
    ],
    compiler_params=pltpu.CompilerParams(use_tc_tiling_on_sc=False),
)
def _sc_embed(uid_i, mid_i, cat_i, mh_i, ch_i, lmap, zrows, uid_t, mid_t,
              cat_t, uid_o, mid_o, cat_o, mids_o, cats_o,
              idx_m, idx_c, row_m, row_c, b_idx, sidx, srow, acc_m, acc_c):
  c = lax.axis_index("c")
  s = lax.axis_index("s")
  wid = s * NC + c
  base = wid * BPW
  arow = s * NPP

  for idx1d, table, out in ((uid_i, uid_t, uid_o),
                            (mid_i, mid_t, mid_o),
                            (cat_i, cat_t, cat_o)):
    pltpu.sync_copy(idx1d.at[pl.ds(base, BPW)], sidx)
    pltpu.sync_copy(table.at[sidx], srow)
    pltpu.sync_copy(srow, out.at[pl.ds(base, BPW)])

  posbase = wid * POS
  for p in range(NP):
    pltpu.sync_copy(zrows, acc_m.at[pl.ds(arow, NPP)])
    pltpu.sync_copy(zrows, acc_c.at[pl.ds(arow, NPP)])

    def _chunk(ci, carry):
      off = posbase + p * POSH + ci * CH
      pltpu.sync_copy(mh_i.at[pl.ds(off, CH)], idx_m)
      pltpu.sync_copy(ch_i.at[pl.ds(off, CH)], idx_c)
      pltpu.sync_copy(lmap.at[pl.ds(s * POSH + ci * CH, CH)], b_idx)
      pltpu.sync_copy(mid_t.at[idx_m], row_m)
      pltpu.sync_copy(cat_t.at[idx_c], row_c)
      pltpu.sync_copy(row_m, acc_m.at[b_idx], add=True)
      pltpu.sync_copy(row_c, acc_c.at[b_idx], add=True)
      return carry
    lax.fori_loop(0, NCH, _chunk, 0)

    pltpu.sync_copy(acc_m.at[pl.ds(arow, NPP)],
                    mids_o.at[pl.ds(base + p * NPP, NPP)])
    pltpu.sync_copy(acc_c.at[pl.ds(arow, NPP)],
                    cats_o.at[pl.ds(base + p * NPP, NPP)])


MB = 512


def _mlp_body(u, m, c, ms, cs, w1, b1, w2, b2, w3, b3, o):
  inp = jnp.concatenate([u[...], m[...], c[...], ms[...], cs[...]], axis=1)
  h = jnp.dot(inp, w1[...], preferred_element_type=_f32) + b1[...]
  h = jnp.maximum(h, 0.0)
  h = jnp.dot(h, w2[...], preferred_element_type=_f32) + b2[...]
  h = jnp.maximum(h, 0.0)
  o[...] = jnp.dot(h, w3[...], preferred_element_type=_f32) + b3[...]


def _mlp(u, m, c, ms, cs, w1p, b1, w2, b2, w3, b3):
  piece = pl.BlockSpec((MB, EP), lambda i: (i, 0))
  full = lambda a: pl.BlockSpec(a.shape, lambda i: (0,) * a.ndim)
  return pl.pallas_call(
      _mlp_body,
      grid=(B // MB,),
      in_specs=[piece] * 5 + [full(w1p), full(b1), full(w2), full(b2),
                              full(w3), full(b3)],
      out_specs=pl.BlockSpec((MB, 1), lambda i: (i, 0)),
      out_shape=jax.ShapeDtypeStruct((B, 1), _f32),
  )(u, m, c, ms, cs, w1p, b1, w2, b2, w3, b3)


def kernel(uid_batch_ph, mid_batch_ph, mid_his_batch_ph, cat_batch_ph,
           cat_his_batch_ph, mask, seq_len_ph, target_ph, lr,
           uid_table, mid_table, cat_table, W1, b1, W2, b2, W3, b3):
  uid_i = uid_batch_ph.astype(_i32)
  mid_i = mid_batch_ph.astype(_i32)
  cat_i = cat_batch_ph.astype(_i32)
  mh_i = mid_his_batch_ph.astype(_i32).reshape(B * L)
  ch_i = cat_his_batch_ph.astype(_i32).reshape(B * L)
  lmap = (jnp.arange(POSH, dtype=_i32) // L)[None, :] \
      + (jnp.arange(NS, dtype=_i32) * NPP)[:, None]
  lmap = lmap.reshape(NS * POSH)
  zrows = jnp.zeros((NPP, EP), _f32)

  pad = ((0, 0), (0, EP - E))
  uid_e, mid_e, cat_e, mids, cats = _sc_embed(
      uid_i, mid_i, cat_i, mh_i, ch_i, lmap, zrows,
      jnp.pad(uid_table.astype(_f32), pad),
      jnp.pad(mid_table.astype(_f32), pad),
      jnp.pad(cat_table.astype(_f32), pad))

  w1p = jnp.zeros((5 * EP, 200), _f32)
  for k in range(5):
    w1p = w1p.at[k * EP:k * EP + E].set(W1[k * E:(k + 1) * E])

  return _mlp(uid_e, mid_e, cat_e, mids, cats,
              w1p, b1.reshape(1, 200), W2, b2.reshape(1, 80),
              W3, b3.reshape(1, 1))


# speedup vs baseline: 8.0348x; 1.1060x over previous
import functools

import jax
import jax.numpy as jnp
from jax import lax
from jax.experimental import pallas as pl
from jax.experimental.pallas import tpu as pltpu
from jax.experimental.pallas import tpu_sc as plsc

B = 16384
L = 200
E = 18
EP = 32
NC = 2
NS = 16
NW = NC * NS
BPW = B // NW
POS = BPW * L
NP = 2
NPP = BPW // NP
POSH = POS // NP
CH = 512
NCH = POSH // CH
NSL = 2

_mesh = plsc.VectorSubcoreMesh(
    core_axis_name="c", subcore_axis_name="s", num_cores=NC, num_subcores=NS)

_f32 = jnp.float32
_i32 = jnp.int32


@functools.partial(
    pl.kernel,
    out_type=[pltpu.HBM((B, EP), _f32)] * 5,
    mesh=_mesh,
    scratch_types=[
        [pltpu.VMEM((CH,), _i32)] * NSL,
        [pltpu.VMEM((CH,), _i32)] * NSL,
        [pltpu.VMEM((CH, EP), _f32)] * NSL,
        [pltpu.VMEM((CH, EP), _f32)] * NSL,
        [pltpu.VMEM((CH,), _i32)] * NSL,
        pltpu.VMEM((BPW,), _i32),
        pltpu.VMEM((BPW, EP), _f32),
        [pltpu.SemaphoreType.DMA] * NSL,
        [pltpu.SemaphoreType.DMA] * NSL,
        [pltpu.SemaphoreType.DMA] * NSL,
        pltpu.VMEM_SHARED((NS * NPP, EP), _f32),
        pltpu.VMEM_SHARED((NS * NPP, EP), _f32),
    ],
    compiler_params=pltpu.CompilerParams(use_tc_tiling_on_sc=False),
)
def _sc_embed(uid_i, mid_i, cat_i, mh_i, ch_i, lmap, zrows, uid_t, mid_t,
              cat_t, uid_o, mid_o, cat_o, mids_o, cats_o,
              idx_m, idx_c, row_m, row_c, b_idx, sidx, srow,
              si, sg, ss, acc_m, acc_c):
  c = lax.axis_index("c")
  s = lax.axis_index("s")
  wid = s * NC + c
  base = wid * BPW
  arow = s * NPP

  for idx1d, table, out in ((uid_i, uid_t, uid_o),
                            (mid_i, mid_t, mid_o),
                            (cat_i, cat_t, cat_o)):
    pltpu.sync_copy(idx1d.at[pl.ds(base, BPW)], sidx)
    pltpu.sync_copy(table.at[sidx], srow)
    pltpu.sync_copy(srow, out.at[pl.ds(base, BPW)])

  posbase = wid * POS

  def _stage_idx(ci, off, b):
    pltpu.async_copy(mh_i.at[pl.ds(off, CH)], idx_m[b], si[b])
    pltpu.async_copy(ch_i.at[pl.ds(off, CH)], idx_c[b], si[b])
    pltpu.async_copy(lmap.at[pl.ds(s * POSH + ci * CH, CH)], b_idx[b], si[b])

  def _wait_idx(b):
    for dst in (idx_m[b], idx_c[b], b_idx[b]):
      pltpu.make_async_copy(mh_i.at[pl.ds(0, CH)], dst, si[b]).wait()

  def _wait_scat(b):
    pltpu.make_async_copy(row_m[b], acc_m.at[b_idx[b]], ss[b]).wait()
    pltpu.make_async_copy(row_c[b], acc_c.at[b_idx[b]], ss[b]).wait()

  for p in range(NP):
    pltpu.sync_copy(zrows, acc_m.at[pl.ds(arow, NPP)])
    pltpu.sync_copy(zrows, acc_c.at[pl.ds(arow, NPP)])
    pbase = posbase + p * POSH
    _stage_idx(0, pbase, 0)

    def _ring(gi, carry):
      for b in range(NSL):
        ci = gi * NSL + b
        nb = 1 - b

        _wait_idx(b)
        gm = pltpu.async_copy(mid_t.at[idx_m[b]], row_m[b], sg[b])
        gc = pltpu.async_copy(cat_t.at[idx_c[b]], row_c[b], sg[b])

        if b == 0:
          @pl.when(gi >= 1)
          def _():
            _wait_scat(nb)
        else:
          _wait_scat(nb)

        if b == NSL - 1:
          @pl.when(gi < NCH // NSL - 1)
          def _():
            _stage_idx(ci + 1, pbase + (ci + 1) * CH, nb)
        else:
          _stage_idx(ci + 1, pbase + (ci + 1) * CH, nb)

        gm.wait()
        gc.wait()
        pltpu.async_copy(row_m[b], acc_m.at[b_idx[b]], ss[b], add=True)
        pltpu.async_copy(row_c[b], acc_c.at[b_idx[b]], ss[b], add=True)
      return carry
    lax.fori_loop(0, NCH // NSL, _ring, 0)
    _wait_scat(NSL - 1)

    pltpu.sync_copy(acc_m.at[pl.ds(arow, NPP)],
                    mids_o.at[pl.ds(base + p * NPP, NPP)])
    pltpu.sync_copy(acc_c.at[pl.ds(arow, NPP)],
                    cats_o.at[pl.ds(base + p * NPP, NPP)])


MB = 512


def _mlp_body(u, m, c, ms, cs, w1, b1, w2, b2, w3, b3, o):
  inp = jnp.concatenate([u[...], m[...], c[...], ms[...], cs[...]], axis=1)
  h = jnp.dot(inp, w1[...], preferred_element_type=_f32) + b1[...]
  h = jnp.maximum(h, 0.0)
  h = jnp.dot(h, w2[...], preferred_element_type=_f32) + b2[...]
  h = jnp.maximum(h, 0.0)
  o[...] = jnp.dot(h, w3[...], preferred_element_type=_f32) + b3[...]


def _mlp(u, m, c, ms, cs, w1p, b1, w2, b2, w3, b3):
  piece = pl.BlockSpec((MB, EP), lambda i: (i, 0))
  full = lambda a: pl.BlockSpec(a.shape, lambda i: (0,) * a.ndim)
  return pl.pallas_call(
      _mlp_body,
      grid=(B // MB,),
      in_specs=[piece] * 5 + [full(w1p), full(b1), full(w2), full(b2),
                              full(w3), full(b3)],
      out_specs=pl.BlockSpec((MB, 1), lambda i: (i, 0)),
      out_shape=jax.ShapeDtypeStruct((B, 1), _f32),
  )(u, m, c, ms, cs, w1p, b1, w2, b2, w3, b3)


def kernel(uid_batch_ph, mid_batch_ph, mid_his_batch_ph, cat_batch_ph,
           cat_his_batch_ph, mask, seq_len_ph, target_ph, lr,
           uid_table, mid_table, cat_table, W1, b1, W2, b2, W3, b3):
  uid_i = uid_batch_ph.astype(_i32)
  mid_i = mid_batch_ph.astype(_i32)
  cat_i = cat_batch_ph.astype(_i32)
  mh_i = mid_his_batch_ph.astype(_i32).reshape(B * L)
  ch_i = cat_his_batch_ph.astype(_i32).reshape(B * L)
  lmap = (jnp.arange(POSH, dtype=_i32) // L)[None, :] \
      + (jnp.arange(NS, dtype=_i32) * NPP)[:, None]
  lmap = lmap.reshape(NS * POSH)
  zrows = jnp.zeros((NPP, EP), _f32)

  pad = ((0, 0), (0, EP - E))
  uid_e, mid_e, cat_e, mids, cats = _sc_embed(
      uid_i, mid_i, cat_i, mh_i, ch_i, lmap, zrows,
      jnp.pad(uid_table.astype(_f32), pad),
      jnp.pad(mid_table.astype(_f32), pad),
      jnp.pad(cat_table.astype(_f32), pad))

  w1p = jnp.zeros((5 * EP, 200), _f32)
  for k in range(5):
    w1p = w1p.at[k * EP:k * EP + E].set(W1[k * E:(k + 1) * E])

  return _mlp(uid_e, mid_e, cat_e, mids, cats,
              w1p, b1.reshape(1, 200), W2, b2.reshape(1, 80),
              W3, b3.reshape(1, 1))
